# baseline (speedup 1.0000x reference)
import functools

import jax
import jax.numpy as jnp
from jax.experimental import pallas as pl
from jax.experimental.pallas import tpu as pltpu

_N = 20000
_NP = 20480
_G = 64
_CH = 512
_NCHUNK = _NP // _CH
_THR = 0.8


def _loss_kernel(nobj_ref, boxes_ref, classes_ref, gt_ref, class_out, coord_out):
    n = nobj_ref[0, 0, 0]

    g = gt_ref[0]
    gx = g[:, 0:1]
    gy = g[:, 1:2]
    gw = g[:, 2:3]
    gh = g[:, 3:4]
    ax1 = gx - gw * 0.5
    ay1 = gy - gh * 0.5
    ax2 = gx + gw * 0.5
    ay2 = gy + gh * 0.5
    area_g = jnp.maximum(ax2 - ax1, 0.0) * jnp.maximum(ay2 - ay1, 0.0)

    col_ids = jax.lax.broadcasted_iota(jnp.int32, (_G, 1), 0)
    valid = col_ids < n

    def chunk_iou(c):
        ds = pl.ds(c * _CH, _CH)
        bx = boxes_ref[0, 0:1, ds]
        by = boxes_ref[0, 1:2, ds]
        bw = boxes_ref[0, 2:3, ds]
        bh = boxes_ref[0, 3:4, ds]
        bx1 = bx - bw * 0.5
        by1 = by - bh * 0.5
        bx2 = bx + bw * 0.5
        by2 = by + bh * 0.5
        w = jnp.maximum(jnp.minimum(ax2, bx2) - jnp.maximum(ax1, bx1), 0.0)
        h = jnp.maximum(jnp.minimum(ay2, by2) - jnp.maximum(ay1, by1), 0.0)
        inter = w * h
        area_b = jnp.maximum(bx2 - bx1, 0.0) * jnp.maximum(by2 - by1, 0.0)
        union = area_g + area_b - inter
        return inter / jnp.maximum(union, 1e-10)

    def p1(c, carry):
        bval, bidx = carry
        iou = chunk_iou(c)
        m = jnp.max(iou, axis=1, keepdims=True)
        lane_ids = jax.lax.broadcasted_iota(jnp.int32, (_G, _CH), 1) + c * _CH
        cand = jnp.min(jnp.where(iou == m, lane_ids, _NP), axis=1, keepdims=True)
        upd = m > bval
        return jnp.where(upd, m, bval), jnp.where(upd, cand, bidx)

    bval0 = jnp.full((_G, 1), -1.0, dtype=jnp.float32)
    bidx0 = jnp.zeros((_G, 1), dtype=jnp.int32)
    _, bidx = jax.lax.fori_loop(0, _NCHUNK, p1, (bval0, bidx0))

    def p2(c, carry):
        coord_acc, class_acc = carry
        iou = chunk_iou(c)
        lane_ids = jax.lax.broadcasted_iota(jnp.int32, (_G, _CH), 1) + c * _CH
        mask = ((iou > _THR) | (lane_ids == bidx)) & valid
        maskf = mask.astype(jnp.float32)

        ds = pl.ds(c * _CH, _CH)
        sl_sum = jnp.zeros((_G, _CH), dtype=jnp.float32)
        for coord in range(4):
            d = boxes_ref[0, coord : coord + 1, ds] - g[:, coord : coord + 1]
            ad = jnp.abs(d)
            sl_sum += jnp.where(ad < 1.0, 0.5 * ad * ad, ad - 0.5)
        coord_acc = coord_acc + jnp.sum(sl_sum * maskf, axis=1, keepdims=True)

        rowpos = jnp.any(mask, axis=0, keepdims=True)
        p0 = classes_ref[0, 0:1, ds]
        p1v = classes_ref[0, 1:2, ds]
        p = jnp.where(rowpos, p1v, p0)
        class_acc = class_acc + (-((1.0 - p) ** 2) * jnp.log(p))
        return coord_acc, class_acc

    coord0 = jnp.zeros((_G, 1), dtype=jnp.float32)
    class0 = jnp.zeros((1, _CH), dtype=jnp.float32)
    coord_acc, class_acc = jax.lax.fori_loop(0, _NCHUNK, p2, (coord0, class0))

    class_out[0, 0, 0] = jnp.sum(class_acc)
    coord_out[0, 0, 0] = jnp.sum(coord_acc)


def kernel(threshhold, batch_boxes, batch_classes, batch_gt, batch_num_objects):
    del threshhold
    B = batch_boxes.shape[0]

    pad = _NP - _N
    boxes_p = jnp.concatenate(
        [
            batch_boxes,
            jnp.broadcast_to(
                jnp.array([4.0, 4.0, 0.0, 0.0], jnp.float32), (B, pad, 4)
            ),
        ],
        axis=1,
    )
    classes_p = jnp.concatenate(
        [batch_classes, jnp.ones((B, pad, 2), jnp.float32)], axis=1
    )
    boxes_t = jnp.transpose(boxes_p, (0, 2, 1))
    classes_t = jnp.transpose(classes_p, (0, 2, 1))
    nobj = batch_num_objects.astype(jnp.int32).reshape(B, 1, 1)

    grid = (B,)
    class_b, coord_b = pl.pallas_call(
        _loss_kernel,
        grid=grid,
        in_specs=[
            pl.BlockSpec((1, 1, 1), lambda b: (b, 0, 0), memory_space=pltpu.SMEM),
            pl.BlockSpec((1, 4, _NP), lambda b: (b, 0, 0)),
            pl.BlockSpec((1, 2, _NP), lambda b: (b, 0, 0)),
            pl.BlockSpec((1, _G, 4), lambda b: (b, 0, 0)),
        ],
        out_specs=[
            pl.BlockSpec((1, 1, 1), lambda b: (b, 0, 0), memory_space=pltpu.SMEM),
            pl.BlockSpec((1, 1, 1), lambda b: (b, 0, 0), memory_space=pltpu.SMEM),
        ],
        out_shape=[
            jax.ShapeDtypeStruct((B, 1, 1), jnp.float32),
            jax.ShapeDtypeStruct((B, 1, 1), jnp.float32),
        ],
        compiler_params=pltpu.CompilerParams(
            dimension_semantics=("arbitrary",),
        ),
    )(nobj, boxes_t, classes_t, batch_gt)

    class_loss = jnp.sum(class_b, axis=(0, 1))
    coord_loss = jnp.sum(coord_b, axis=(0, 1))
    total = class_loss + coord_loss
    return (total, class_loss, coord_loss)
